# baseline (speedup 1.0000x reference)
import functools

import jax
import jax.numpy as jnp
from jax.experimental import pallas as pl
from jax.experimental.pallas import tpu as pltpu


def _router_body(x_ref, wcat_ref, bg_ref, be_ref, out_ref, *, E, H):
    xb = x_ref[...].astype(jnp.bfloat16)
    y = jnp.dot(xb, wcat_ref[...], preferred_element_type=jnp.float32)

    logits = y[:, :E] + bg_ref[...]
    m = jnp.max(logits, axis=-1, keepdims=True)
    ex = jnp.exp(logits - m)
    probs = ex / jnp.sum(ex, axis=-1, keepdims=True)

    eidx = jax.lax.broadcasted_iota(jnp.int32, probs.shape, 1)
    m1 = jnp.max(probs, axis=-1, keepdims=True)
    idx1 = jnp.min(jnp.where(probs == m1, eidx, E), axis=-1, keepdims=True)
    probs_m = jnp.where(eidx == idx1, -jnp.inf, probs)
    m2 = jnp.max(probs_m, axis=-1, keepdims=True)
    idx2 = jnp.min(jnp.where(probs_m == m2, eidx, E), axis=-1, keepdims=True)

    s = m1 + m2
    inv = 1.0 / (s + 1e-8)
    w1 = jnp.where(s <= 0, 0.5, m1 * inv)
    w2 = jnp.where(s <= 0, 0.5, m2 * inv)
    c = w1 * (eidx == idx1).astype(jnp.float32) + w2 * (eidx == idx2).astype(jnp.float32)

    acc = jnp.dot(c, be_ref[...], preferred_element_type=jnp.float32)
    for e in range(E):
        acc = acc + c[:, e:e + 1] * y[:, 128 + e * H:128 + (e + 1) * H]
    out_ref[...] = acc


def kernel(x, Wg, bg, We, be, context_length, horizon, top_k):
    B, C = x.shape
    E, _, H = We.shape
    W_all = jnp.transpose(We, (1, 0, 2)).reshape(C, E * H)
    Wg_pad = jnp.pad(Wg, ((0, 0), (0, 128 - E)))
    W_cat = jnp.concatenate([Wg_pad, W_all], axis=1).astype(jnp.bfloat16)
    bg2 = bg.reshape(1, E)
    N = 128 + E * H

    Bb = 1024
    grid = (B // Bb,)
    body = functools.partial(_router_body, E=E, H=H)
    return pl.pallas_call(
        body,
        grid=grid,
        in_specs=[
            pl.BlockSpec((Bb, C), lambda i: (i, 0)),
            pl.BlockSpec((C, N), lambda i: (0, 0)),
            pl.BlockSpec((1, E), lambda i: (0, 0)),
            pl.BlockSpec((E, H), lambda i: (0, 0)),
        ],
        out_specs=pl.BlockSpec((Bb, H), lambda i: (i, 0)),
        out_shape=jax.ShapeDtypeStruct((B, H), jnp.float32),
        compiler_params=pltpu.CompilerParams(
            dimension_semantics=("parallel",)),
    )(x, W_cat, bg2, be)


# speedup vs baseline: 1.0461x; 1.0461x over previous
import functools

import jax
import jax.numpy as jnp
from jax.experimental import pallas as pl
from jax.experimental.pallas import tpu as pltpu


def _router_body(x_ref, wcat_hbm, bg_ref, be_ref, out_ref, wcat_vmem, sem,
                 *, E, H):
    @pl.when(pl.program_id(0) == 0)
    def _():
        copy = pltpu.make_async_copy(wcat_hbm, wcat_vmem, sem)
        copy.start()
        copy.wait()

    xb = x_ref[...].astype(jnp.bfloat16)
    y = jnp.dot(xb, wcat_vmem[...], preferred_element_type=jnp.float32)

    logits = y[:, :E] + bg_ref[...]
    m = jnp.max(logits, axis=-1, keepdims=True)
    ex = jnp.exp(logits - m)
    probs = ex / jnp.sum(ex, axis=-1, keepdims=True)

    eidx = jax.lax.broadcasted_iota(jnp.int32, probs.shape, 1)
    m1 = jnp.max(probs, axis=-1, keepdims=True)
    idx1 = jnp.min(jnp.where(probs == m1, eidx, E), axis=-1, keepdims=True)
    probs_m = jnp.where(eidx == idx1, -jnp.inf, probs)
    m2 = jnp.max(probs_m, axis=-1, keepdims=True)
    idx2 = jnp.min(jnp.where(probs_m == m2, eidx, E), axis=-1, keepdims=True)

    s = m1 + m2
    inv = 1.0 / (s + 1e-8)
    w1 = jnp.where(s <= 0, 0.5, m1 * inv)
    w2 = jnp.where(s <= 0, 0.5, m2 * inv)
    c = w1 * (eidx == idx1).astype(jnp.float32) + w2 * (eidx == idx2).astype(jnp.float32)

    acc = jnp.dot(c, be_ref[...], preferred_element_type=jnp.float32)
    for e in range(E):
        acc = acc + c[:, e:e + 1] * y[:, 128 + e * H:128 + (e + 1) * H]
    out_ref[...] = acc


def kernel(x, Wg, bg, We, be, context_length, horizon, top_k):
    B, C = x.shape
    E, _, H = We.shape
    W_all = jnp.transpose(We, (1, 0, 2)).reshape(C, E * H)
    Wg_pad = jnp.pad(Wg, ((0, 0), (0, 128 - E)))
    W_cat = jnp.concatenate([Wg_pad, W_all], axis=1).astype(jnp.bfloat16)
    bg2 = bg.reshape(1, E)
    N = 128 + E * H

    Bb = 512
    grid = (B // Bb,)
    body = functools.partial(_router_body, E=E, H=H)
    return pl.pallas_call(
        body,
        grid=grid,
        in_specs=[
            pl.BlockSpec((Bb, C), lambda i: (i, 0)),
            pl.BlockSpec(memory_space=pl.ANY),
            pl.BlockSpec((1, E), lambda i: (0, 0)),
            pl.BlockSpec((E, H), lambda i: (0, 0)),
        ],
        out_specs=pl.BlockSpec((Bb, H), lambda i: (i, 0)),
        out_shape=jax.ShapeDtypeStruct((B, H), jnp.float32),
        scratch_shapes=[
            pltpu.VMEM((C, N), jnp.bfloat16),
            pltpu.SemaphoreType.DMA,
        ],
        compiler_params=pltpu.CompilerParams(
            dimension_semantics=("arbitrary",)),
    )(x, W_cat, bg2, be)


# speedup vs baseline: 1.1543x; 1.1035x over previous
import functools

import jax
import jax.numpy as jnp
from jax.experimental import pallas as pl
from jax.experimental.pallas import tpu as pltpu


def _router_body(x_ref, wg_ref, bg_ref, wall_ref, be_ref, out_ref, *, E, H):
    xb = x_ref[...].astype(jnp.bfloat16)
    logits = jnp.dot(xb, wg_ref[...], preferred_element_type=jnp.float32)
    logits = logits + bg_ref[...]
    m = jnp.max(logits, axis=-1, keepdims=True)
    ex = jnp.exp(logits - m)
    probs = ex / jnp.sum(ex, axis=-1, keepdims=True)

    eidx = jax.lax.broadcasted_iota(jnp.int32, probs.shape, 1)
    m1 = jnp.max(probs, axis=-1, keepdims=True)
    idx1 = jnp.min(jnp.where(probs == m1, eidx, E), axis=-1, keepdims=True)
    probs_m = jnp.where(eidx == idx1, -jnp.inf, probs)
    m2 = jnp.max(probs_m, axis=-1, keepdims=True)
    idx2 = jnp.min(jnp.where(probs_m == m2, eidx, E), axis=-1, keepdims=True)

    s = m1 + m2
    inv = 1.0 / (s + 1e-8)
    w1 = jnp.where(s <= 0, 0.5, m1 * inv)
    w2 = jnp.where(s <= 0, 0.5, m2 * inv)
    c = w1 * (eidx == idx1).astype(jnp.float32) + w2 * (eidx == idx2).astype(jnp.float32)

    preds = jnp.dot(xb, wall_ref[...], preferred_element_type=jnp.float32)

    acc = jnp.dot(c, be_ref[...], preferred_element_type=jnp.float32)
    for e in range(E):
        acc = acc + c[:, e:e + 1] * preds[:, e * H:(e + 1) * H]
    out_ref[...] = acc


def kernel(x, Wg, bg, We, be, context_length, horizon, top_k):
    B, C = x.shape
    E, _, H = We.shape
    W_all = jnp.transpose(We, (1, 0, 2)).reshape(C, E * H).astype(jnp.bfloat16)
    Wg_bf = Wg.astype(jnp.bfloat16)
    bg2 = bg.reshape(1, E)

    Bb = 256
    grid = (B // Bb,)
    body = functools.partial(_router_body, E=E, H=H)
    return pl.pallas_call(
        body,
        grid=grid,
        in_specs=[
            pl.BlockSpec((Bb, C), lambda i: (i, 0)),
            pl.BlockSpec((C, E), lambda i: (0, 0)),
            pl.BlockSpec((1, E), lambda i: (0, 0)),
            pl.BlockSpec((C, E * H), lambda i: (0, 0)),
            pl.BlockSpec((E, H), lambda i: (0, 0)),
        ],
        out_specs=pl.BlockSpec((Bb, H), lambda i: (i, 0)),
        out_shape=jax.ShapeDtypeStruct((B, H), jnp.float32),
        compiler_params=pltpu.CompilerParams(
            dimension_semantics=("arbitrary",)),
    )(x, Wg_bf, bg2, W_all, be)


# speedup vs baseline: 1.3865x; 1.2011x over previous
import functools

import jax
import jax.numpy as jnp
from jax.experimental import pallas as pl
from jax.experimental.pallas import tpu as pltpu


def _router_body(x_ref, wg_ref, bg_ref, wall_ref, be_ref, out_ref, *, E, H):
    xb = x_ref[...].astype(jnp.bfloat16)
    logits = jnp.dot(xb, wg_ref[...], preferred_element_type=jnp.float32)
    logits = logits + bg_ref[...]
    lt = logits.T
    m = jnp.max(lt, axis=0, keepdims=True)
    ex = jnp.exp(lt - m)
    probs = ex / jnp.sum(ex, axis=0, keepdims=True)

    eidx = jax.lax.broadcasted_iota(jnp.int32, probs.shape, 0)
    m1 = jnp.max(probs, axis=0, keepdims=True)
    idx1 = jnp.min(jnp.where(probs == m1, eidx, E), axis=0, keepdims=True)
    probs_m = jnp.where(eidx == idx1, -jnp.inf, probs)
    m2 = jnp.max(probs_m, axis=0, keepdims=True)
    idx2 = jnp.min(jnp.where(probs_m == m2, eidx, E), axis=0, keepdims=True)

    s = m1 + m2
    inv = 1.0 / (s + 1e-8)
    w1 = jnp.where(s <= 0, 0.5, m1 * inv)
    w2 = jnp.where(s <= 0, 0.5, m2 * inv)
    cT = w1 * (eidx == idx1).astype(jnp.float32) + w2 * (eidx == idx2).astype(jnp.float32)
    c = cT.T

    preds = jnp.dot(xb, wall_ref[...], preferred_element_type=jnp.float32)

    acc = jnp.dot(c, be_ref[...], preferred_element_type=jnp.float32)
    for e in range(E):
        acc = acc + c[:, e:e + 1] * preds[:, e * H:(e + 1) * H]
    out_ref[...] = acc


def kernel(x, Wg, bg, We, be, context_length, horizon, top_k):
    B, C = x.shape
    E, _, H = We.shape
    W_all = jnp.transpose(We, (1, 0, 2)).reshape(C, E * H).astype(jnp.bfloat16)
    Wg_bf = Wg.astype(jnp.bfloat16)
    bg2 = bg.reshape(1, E)

    Bb = 512
    grid = (B // Bb,)
    body = functools.partial(_router_body, E=E, H=H)
    return pl.pallas_call(
        body,
        grid=grid,
        in_specs=[
            pl.BlockSpec((Bb, C), lambda i: (i, 0)),
            pl.BlockSpec((C, E), lambda i: (0, 0)),
            pl.BlockSpec((1, E), lambda i: (0, 0)),
            pl.BlockSpec((C, E * H), lambda i: (0, 0)),
            pl.BlockSpec((E, H), lambda i: (0, 0)),
        ],
        out_specs=pl.BlockSpec((Bb, H), lambda i: (i, 0)),
        out_shape=jax.ShapeDtypeStruct((B, H), jnp.float32),
        compiler_params=pltpu.CompilerParams(
            dimension_semantics=("arbitrary",)),
    )(x, Wg_bf, bg2, W_all, be)
